# baseline (speedup 1.0000x reference)
import jax
import jax.numpy as jnp
from jax.experimental import pallas as pl

_C = 768
_E = 8
_DFF = 3072
_T = 2048
_CAP = 256
_TBLK = 256


def _router_body(x_ref, wc_ref, bc_ref,
                 s0_ref, e0_ref, e1_ref, w0_ref, w1_ref, st_ref):
    x = x_ref[...]
    scores = jnp.dot(x, wc_ref[...], preferred_element_type=jnp.float32)
    scores = jax.nn.sigmoid(scores + bc_ref[...])
    eidx = jax.lax.broadcasted_iota(jnp.int32, (_T, _E), 1)
    g0 = jnp.max(scores, axis=1, keepdims=True)
    e0 = jnp.min(jnp.where(scores == g0, eidx, _E), axis=1, keepdims=True)
    sc2 = jnp.where(eidx == e0, -jnp.inf, scores)
    g1 = jnp.max(sc2, axis=1, keepdims=True)
    e1 = jnp.min(jnp.where(sc2 == g1, eidx, _E), axis=1, keepdims=True)
    denom = g0 + g1
    s0_ref[...] = g0 / denom
    e0_ref[...] = e0
    e1_ref[...] = e1
    w0_ref[...] = g0 / denom
    w1_ref[...] = g1 / denom
    cnt = jnp.sum((eidx == e0).astype(jnp.float32)
                  + (eidx == e1).astype(jnp.float32), axis=0, keepdims=True)
    p = jnp.sum(scores / jnp.sum(scores, axis=1, keepdims=True),
                axis=0, keepdims=True) / _T
    st_ref[...] = jnp.concatenate([cnt / _T, p], axis=0)


def _pos_body(s0c_ref, e0c_ref, e1c_ref, s0r_ref, e0r_ref, e1r_ref,
              w0c_ref, w1c_ref,
              ds0_ref, ds1_ref, cs0_ref, cs1_ref, w0e_ref, w1e_ref):
    i = pl.program_id(0)
    s0c = s0c_ref[...]
    s0r = s0r_ref[...]
    e0c, e1c = e0c_ref[...], e1c_ref[...]
    e0r, e1r = e0r_ref[...], e1r_ref[...]
    il = i * _TBLK + jax.lax.broadcasted_iota(jnp.int32, (_TBLK, 1), 0)
    jl = jax.lax.broadcasted_iota(jnp.int32, (1, _T), 1)
    r = (s0r > s0c) | ((s0r == s0c) & (jl <= il))
    pos0 = jnp.sum((r & (e0r == e0c)).astype(jnp.int32), axis=1, keepdims=True)
    cnt0 = jnp.sum((e0r == e1c).astype(jnp.int32), axis=1, keepdims=True)
    pos1 = cnt0 + jnp.sum((r & (e1r == e1c)).astype(jnp.int32),
                          axis=1, keepdims=True)
    v0 = pos0 <= _CAP
    v1 = pos1 <= _CAP
    slot0 = e0c * _CAP + pos0 - 1
    slot1 = e1c * _CAP + pos1 - 1
    ds0_ref[...] = jnp.where(v0, slot0, _T)
    ds1_ref[...] = jnp.where(v1, slot1, _T)
    cs0_ref[...] = jnp.where(v0, slot0, 0)
    cs1_ref[...] = jnp.where(v1, slot1, 0)
    w0e_ref[...] = jnp.where(v0, w0c_ref[...], 0.0)
    w1e_ref[...] = jnp.where(v1, w1c_ref[...], 0.0)


def _ffn_body(x_ref, ds0_ref, ds1_ref, w1_ref, b1_ref, w2_ref, b2_ref,
              out_ref):
    e = pl.program_id(0)
    sl = e * _CAP + jax.lax.broadcasted_iota(jnp.int32, (_CAP, _T), 0)
    ds0 = ds0_ref[...]
    ds1 = ds1_ref[...]
    mask = ((ds0 == sl) | (ds1 == sl)).astype(jnp.bfloat16)
    xin = jnp.dot(mask, x_ref[...], preferred_element_type=jnp.float32)
    h = jnp.dot(xin.astype(jnp.bfloat16), w1_ref[0],
                preferred_element_type=jnp.float32)
    h = jax.nn.gelu(h + b1_ref[0])
    out = jnp.dot(h.astype(jnp.bfloat16), w2_ref[0],
                  preferred_element_type=jnp.float32)
    out_ref[...] = (out + b2_ref[0]).astype(jnp.bfloat16)


def _combine_body(cs0_ref, cs1_ref, w0e_ref, w1e_ref, eo_ref, out_ref):
    sl = jax.lax.broadcasted_iota(jnp.int32, (_TBLK, _T), 1)
    wmask = (jnp.where(cs0_ref[...] == sl, w0e_ref[...], 0.0)
             + jnp.where(cs1_ref[...] == sl, w1e_ref[...], 0.0))
    out_ref[...] = jnp.dot(wmask.astype(jnp.bfloat16), eo_ref[...],
                           preferred_element_type=jnp.float32)


def kernel(x, Ws, bs, Wc, bc, W1, b1, W2, b2):
    del Ws, bs
    xf = x.reshape(_T, _C)
    f32 = jnp.float32
    i32 = jnp.int32
    col_f = jax.ShapeDtypeStruct((_T, 1), f32)
    col_i = jax.ShapeDtypeStruct((_T, 1), i32)

    s0, e0, e1, w0, w1, st = pl.pallas_call(
        _router_body,
        out_shape=(col_f, col_i, col_i, col_f, col_f,
                   jax.ShapeDtypeStruct((2, _E), f32)),
    )(xf, Wc, bc.reshape(1, _E))

    cblk_f = pl.BlockSpec((_TBLK, 1), lambda i: (i, 0))
    cblk_i = pl.BlockSpec((_TBLK, 1), lambda i: (i, 0))
    row_f = pl.BlockSpec((1, _T), lambda i: (0, 0))
    row_i = pl.BlockSpec((1, _T), lambda i: (0, 0))
    ds0, ds1, cs0, cs1, w0e, w1e = pl.pallas_call(
        _pos_body,
        grid=(_T // _TBLK,),
        in_specs=[cblk_f, cblk_i, cblk_i, row_f, row_i, row_i, cblk_f, cblk_f],
        out_specs=(cblk_i, cblk_i, cblk_i, cblk_i, cblk_f, cblk_f),
        out_shape=(col_i, col_i, col_i, col_i, col_f, col_f),
    )(s0, e0, e1,
      s0.reshape(1, _T), e0.reshape(1, _T), e1.reshape(1, _T), w0, w1)

    bf16 = jnp.bfloat16
    eo = pl.pallas_call(
        _ffn_body,
        grid=(_E,),
        in_specs=[
            pl.BlockSpec((_T, _C), lambda e: (0, 0)),
            pl.BlockSpec((1, _T), lambda e: (0, 0)),
            pl.BlockSpec((1, _T), lambda e: (0, 0)),
            pl.BlockSpec((1, _C, _DFF), lambda e: (e, 0, 0)),
            pl.BlockSpec((1, 1, _DFF), lambda e: (e, 0, 0)),
            pl.BlockSpec((1, _DFF, _C), lambda e: (e, 0, 0)),
            pl.BlockSpec((1, 1, _C), lambda e: (e, 0, 0)),
        ],
        out_specs=pl.BlockSpec((_CAP, _C), lambda e: (e, 0)),
        out_shape=jax.ShapeDtypeStruct((_E * _CAP, _C), bf16),
    )(xf.astype(bf16), ds0.reshape(1, _T), ds1.reshape(1, _T),
      W1.astype(bf16), b1.reshape(_E, 1, _DFF), W2.astype(bf16),
      b2.reshape(_E, 1, _C))

    out = pl.pallas_call(
        _combine_body,
        grid=(_T // _TBLK,),
        in_specs=[cblk_i, cblk_i, cblk_f, cblk_f,
                  pl.BlockSpec((_E * _CAP, _C), lambda i: (0, 0))],
        out_specs=pl.BlockSpec((_TBLK, _C), lambda i: (i, 0)),
        out_shape=jax.ShapeDtypeStruct((_T, _C), f32),
    )(cs0, cs1, w0e, w1e, eo)

    tpe = st[0]
    p = st[1]
    return out.reshape(x.shape), tpe, tpe, p


# speedup vs baseline: 1.5321x; 1.5321x over previous
import jax
import jax.numpy as jnp
from jax.experimental import pallas as pl

_C = 768
_E = 8
_DFF = 3072
_T = 2048
_CAP = 256
_TBLK = 256


def _router_body(x_ref, wc_ref, bc_ref,
                 s0_ref, e0_ref, e1_ref, w0_ref, w1_ref, st_ref):
    x = x_ref[...]
    scores = jnp.dot(x, wc_ref[...], preferred_element_type=jnp.float32)
    scores = jax.nn.sigmoid(scores + bc_ref[...])
    eidx = jax.lax.broadcasted_iota(jnp.int32, (_T, _E), 1)
    g0 = jnp.max(scores, axis=1, keepdims=True)
    e0 = jnp.min(jnp.where(scores == g0, eidx, _E), axis=1, keepdims=True)
    sc2 = jnp.where(eidx == e0, -jnp.inf, scores)
    g1 = jnp.max(sc2, axis=1, keepdims=True)
    e1 = jnp.min(jnp.where(sc2 == g1, eidx, _E), axis=1, keepdims=True)
    denom = g0 + g1
    s0_ref[...] = g0 / denom
    e0_ref[...] = e0
    e1_ref[...] = e1
    w0_ref[...] = g0 / denom
    w1_ref[...] = g1 / denom
    cnt = jnp.sum((eidx == e0).astype(jnp.float32)
                  + (eidx == e1).astype(jnp.float32), axis=0, keepdims=True)
    p = jnp.sum(scores / jnp.sum(scores, axis=1, keepdims=True),
                axis=0, keepdims=True) / _T
    st_ref[...] = jnp.concatenate([cnt / _T, p], axis=0)


def _pos_body(s0c_ref, e0c_ref, e1c_ref, s0r_ref, e0r_ref, e1r_ref,
              w0c_ref, w1c_ref,
              ds0_ref, ds1_ref, cs0_ref, cs1_ref, w0e_ref, w1e_ref):
    i = pl.program_id(0)
    s0c = s0c_ref[...]
    s0r = s0r_ref[...]
    e0c, e1c = e0c_ref[...], e1c_ref[...]
    e0r, e1r = e0r_ref[...], e1r_ref[...]
    il = i * _TBLK + jax.lax.broadcasted_iota(jnp.int32, (_TBLK, 1), 0)
    jl = jax.lax.broadcasted_iota(jnp.int32, (1, _T), 1)
    r = (s0r > s0c) | ((s0r == s0c) & (jl <= il))
    pos0 = jnp.sum((r & (e0r == e0c)).astype(jnp.int32), axis=1, keepdims=True)
    cnt0 = jnp.sum((e0r == e1c).astype(jnp.int32), axis=1, keepdims=True)
    pos1 = cnt0 + jnp.sum((r & (e1r == e1c)).astype(jnp.int32),
                          axis=1, keepdims=True)
    v0 = pos0 <= _CAP
    v1 = pos1 <= _CAP
    slot0 = e0c * _CAP + pos0 - 1
    slot1 = e1c * _CAP + pos1 - 1
    ds0_ref[...] = jnp.where(v0, slot0, _T)
    ds1_ref[...] = jnp.where(v1, slot1, _T)
    cs0_ref[...] = jnp.where(v0, slot0, 0)
    cs1_ref[...] = jnp.where(v1, slot1, 0)
    w0e_ref[...] = jnp.where(v0, w0c_ref[...], 0.0)
    w1e_ref[...] = jnp.where(v1, w1c_ref[...], 0.0)


def _ffn_body(x_ref, ds0_ref, ds1_ref, w1_ref, b1_ref, w2_ref, b2_ref,
              out_ref):
    e = pl.program_id(0)
    sl = e * _CAP + jax.lax.broadcasted_iota(jnp.int32, (_CAP, _T), 0)
    ds0 = ds0_ref[...]
    ds1 = ds1_ref[...]
    mask = ((ds0 == sl) | (ds1 == sl)).astype(jnp.bfloat16)
    xin = jnp.dot(mask, x_ref[...].astype(jnp.bfloat16),
                  preferred_element_type=jnp.float32)
    h = jnp.dot(xin.astype(jnp.bfloat16), w1_ref[0].astype(jnp.bfloat16),
                preferred_element_type=jnp.float32)
    h = jax.nn.gelu(h + b1_ref[0])
    out = jnp.dot(h.astype(jnp.bfloat16), w2_ref[0].astype(jnp.bfloat16),
                  preferred_element_type=jnp.float32)
    out_ref[...] = (out + b2_ref[0]).astype(jnp.bfloat16)


def _combine_body(cs0_ref, cs1_ref, w0e_ref, w1e_ref, eo_ref, out_ref):
    sl = jax.lax.broadcasted_iota(jnp.int32, (_TBLK, _T), 1)
    wmask = (jnp.where(cs0_ref[...] == sl, w0e_ref[...], 0.0)
             + jnp.where(cs1_ref[...] == sl, w1e_ref[...], 0.0))
    out_ref[...] = jnp.dot(wmask.astype(jnp.bfloat16), eo_ref[...],
                           preferred_element_type=jnp.float32)


def kernel(x, Ws, bs, Wc, bc, W1, b1, W2, b2):
    del Ws, bs
    xf = x.reshape(_T, _C)
    f32 = jnp.float32
    i32 = jnp.int32
    col_f = jax.ShapeDtypeStruct((_T, 1), f32)
    col_i = jax.ShapeDtypeStruct((_T, 1), i32)

    s0, e0, e1, w0, w1, st = pl.pallas_call(
        _router_body,
        out_shape=(col_f, col_i, col_i, col_f, col_f,
                   jax.ShapeDtypeStruct((2, _E), f32)),
    )(xf, Wc, bc.reshape(1, _E))

    cblk_f = pl.BlockSpec((_TBLK, 1), lambda i: (i, 0))
    cblk_i = pl.BlockSpec((_TBLK, 1), lambda i: (i, 0))
    row_f = pl.BlockSpec((1, _T), lambda i: (0, 0))
    row_i = pl.BlockSpec((1, _T), lambda i: (0, 0))
    ds0, ds1, cs0, cs1, w0e, w1e = pl.pallas_call(
        _pos_body,
        grid=(_T // _TBLK,),
        in_specs=[cblk_f, cblk_i, cblk_i, row_f, row_i, row_i, cblk_f, cblk_f],
        out_specs=(cblk_i, cblk_i, cblk_i, cblk_i, cblk_f, cblk_f),
        out_shape=(col_i, col_i, col_i, col_i, col_f, col_f),
    )(s0, e0, e1,
      s0.reshape(1, _T), e0.reshape(1, _T), e1.reshape(1, _T), w0, w1)

    eo = pl.pallas_call(
        _ffn_body,
        grid=(_E,),
        in_specs=[
            pl.BlockSpec((_T, _C), lambda e: (0, 0)),
            pl.BlockSpec((1, _T), lambda e: (0, 0)),
            pl.BlockSpec((1, _T), lambda e: (0, 0)),
            pl.BlockSpec((1, _C, _DFF), lambda e: (e, 0, 0)),
            pl.BlockSpec((1, 1, _DFF), lambda e: (e, 0, 0)),
            pl.BlockSpec((1, _DFF, _C), lambda e: (e, 0, 0)),
            pl.BlockSpec((1, 1, _C), lambda e: (e, 0, 0)),
        ],
        out_specs=pl.BlockSpec((_CAP, _C), lambda e: (e, 0)),
        out_shape=jax.ShapeDtypeStruct((_E * _CAP, _C), jnp.bfloat16),
    )(xf, ds0.reshape(1, _T), ds1.reshape(1, _T),
      W1, b1.reshape(_E, 1, _DFF), W2, b2.reshape(_E, 1, _C))

    out = pl.pallas_call(
        _combine_body,
        grid=(_T // _TBLK,),
        in_specs=[cblk_i, cblk_i, cblk_f, cblk_f,
                  pl.BlockSpec((_E * _CAP, _C), lambda i: (0, 0))],
        out_specs=pl.BlockSpec((_TBLK, _C), lambda i: (i, 0)),
        out_shape=jax.ShapeDtypeStruct((_T, _C), f32),
    )(cs0, cs1, w0e, w1e, eo)

    tpe = st[0]
    p = st[1]
    return out.reshape(x.shape), tpe, tpe, p
